# baseline (speedup 1.0000x reference)
import jax
import jax.numpy as jnp
from jax.experimental import pallas as pl
from jax.experimental.pallas import tpu as pltpu

BATCH = 4096
N_CLUSTERS = 8192
BL = 1024
SG = 8


def _assign_kernel(x_ref, c_ref, out_ref):
    x0 = x_ref[0:1, :]
    x1 = x_ref[1:2, :]

    def body(t, carry):
        bestv, bidx = carry
        c = c_ref[pl.ds(t * SG, SG), :]
        c0 = c[:, 0:1]
        c1 = c[:, 1:2]
        d0 = x0 - c0
        d1 = x1 - c1
        dist = d0 * d0 + d1 * d1
        mask = dist < bestv
        bestv = jnp.where(mask, dist, bestv)
        bidx = jnp.where(mask, t, bidx)
        return bestv, bidx

    bestv0 = jnp.full((SG, BL), jnp.inf, dtype=jnp.float32)
    bidx0 = jnp.zeros((SG, BL), dtype=jnp.int32)
    bestv, bidx = jax.lax.fori_loop(0, N_CLUSTERS // SG, body, (bestv0, bidx0))

    m = jnp.min(bestv, axis=0, keepdims=True)
    srow = jax.lax.broadcasted_iota(jnp.int32, (SG, BL), 0)
    cand = jnp.where(bestv == m, bidx * SG + srow, N_CLUSTERS)
    out_ref[:] = jnp.min(cand, axis=0)


def kernel(inputs, cluster_centers):
    inputs_t = inputs.T
    grid = (BATCH // BL,)
    return pl.pallas_call(
        _assign_kernel,
        grid=grid,
        in_specs=[
            pl.BlockSpec((2, BL), lambda i: (0, i)),
            pl.BlockSpec((N_CLUSTERS, 2), lambda i: (0, 0)),
        ],
        out_specs=pl.BlockSpec((BL,), lambda i: (i,)),
        out_shape=jax.ShapeDtypeStruct((BATCH,), jnp.int32),
        compiler_params=pltpu.CompilerParams(
            dimension_semantics=("parallel",),
        ),
    )(inputs_t, cluster_centers)


# speedup vs baseline: 4.3286x; 4.3286x over previous
import jax
import jax.numpy as jnp
from jax.experimental import pallas as pl
from jax.experimental.pallas import tpu as pltpu

BATCH = 4096
N_CLUSTERS = 8192
R = 512
RG = 128
CK = 128


def _assign_kernel(x_ref, c_ref, out_ref):
    n_chunks = N_CLUSTERS // CK
    lane = jax.lax.broadcasted_iota(jnp.int32, (RG, CK), 1)

    def row_group(g, _):
        x0 = x_ref[pl.ds(g * RG, RG), 0:1]
        x1 = x_ref[pl.ds(g * RG, RG), 1:2]
        x0b = jnp.broadcast_to(x0, (RG, CK))
        x1b = jnp.broadcast_to(x1, (RG, CK))

        def body(t, carry):
            bestv, bidx = carry
            c0 = c_ref[0:1, pl.ds(t * CK, CK)]
            c1 = c_ref[1:2, pl.ds(t * CK, CK)]
            d0 = x0b - c0
            d1 = x1b - c1
            dist = d0 * d0 + d1 * d1
            mask = dist < bestv
            bestv = jnp.where(mask, dist, bestv)
            bidx = jnp.where(mask, t, bidx)
            return bestv, bidx

        bestv0 = jnp.full((RG, CK), jnp.inf, dtype=jnp.float32)
        bidx0 = jnp.zeros((RG, CK), dtype=jnp.int32)
        bestv, bidx = jax.lax.fori_loop(0, n_chunks, body, (bestv0, bidx0))

        m = jnp.min(bestv, axis=-1, keepdims=True)
        cand = jnp.where(bestv == m, bidx * CK + lane, N_CLUSTERS)
        out_ref[pl.ds(g * RG, RG)] = jnp.min(cand, axis=-1)
        return 0

    jax.lax.fori_loop(0, R // RG, row_group, 0)


def kernel(inputs, cluster_centers):
    centers_t = cluster_centers.T
    grid = (BATCH // R,)
    return pl.pallas_call(
        _assign_kernel,
        grid=grid,
        in_specs=[
            pl.BlockSpec((R, 2), lambda i: (i, 0)),
            pl.BlockSpec((2, N_CLUSTERS), lambda i: (0, 0)),
        ],
        out_specs=pl.BlockSpec((R,), lambda i: (i,)),
        out_shape=jax.ShapeDtypeStruct((BATCH,), jnp.int32),
        compiler_params=pltpu.CompilerParams(
            dimension_semantics=("parallel",),
        ),
    )(inputs, centers_t)


# speedup vs baseline: 4.7883x; 1.1062x over previous
import jax
import jax.numpy as jnp
from jax.experimental import pallas as pl
from jax.experimental.pallas import tpu as pltpu

BATCH = 4096
N_CLUSTERS = 8192
B_TILE = 512


def _assign_kernel(x_ref, c_ref, out_ref):
    x0 = x_ref[:, 0:1]
    x1 = x_ref[:, 1:2]
    c0 = c_ref[0:1, :]
    c1 = c_ref[1:2, :]
    d0 = x0 - c0
    d1 = x1 - c1
    dist = d0 * d0 + d1 * d1
    out_ref[:] = jnp.argmin(dist, axis=-1).astype(jnp.int32)


def kernel(inputs, cluster_centers):
    centers_t = cluster_centers.T
    grid = (BATCH // B_TILE,)
    return pl.pallas_call(
        _assign_kernel,
        grid=grid,
        in_specs=[
            pl.BlockSpec((B_TILE, 2), lambda i: (i, 0)),
            pl.BlockSpec((2, N_CLUSTERS), lambda i: (0, 0)),
        ],
        out_specs=pl.BlockSpec((B_TILE,), lambda i: (i,)),
        out_shape=jax.ShapeDtypeStruct((BATCH,), jnp.int32),
        compiler_params=pltpu.CompilerParams(
            dimension_semantics=("parallel",),
        ),
    )(inputs, centers_t)
